# baseline (speedup 1.0000x reference)
import jax
import jax.numpy as jnp
from jax import lax
from jax.experimental import pallas as pl
from jax.experimental.pallas import tpu as pltpu

_CAPACITY = 16384
_KEY_BITS = 2048
_VALUE_BITS = 2048
_BLK = 512
_NBLK = _CAPACITY // _BLK
_NBUF = 4
_HALF = _NBLK // 2


def _blk_of(k):
    return (k // 2) + (k % 2) * _HALF


def _blk_start(keys_hbm, kbuf, sems, k, b):
    copy = pltpu.make_async_copy(
        keys_hbm.at[pl.ds(_blk_of(k) * _BLK, _BLK)], kbuf.at[b], sems[b]
    )
    copy.start()


def _body(q_ref, keys_hbm, values_hbm, out_ref, kbuf, best_ref, gsem, *sems):
    for b in range(_NBUF):
        _blk_start(keys_hbm, kbuf, sems, b, b)
    best_ref[0] = jnp.int32(2**30)

    def super_body(s, _):
        for b in range(_NBUF):
            k = s * _NBUF + b
            blk = _blk_of(k)
            pltpu.make_async_copy(
                keys_hbm.at[pl.ds(blk * _BLK, _BLK)], kbuf.at[b], sems[b]
            ).wait()
            xor = jnp.bitwise_xor(kbuf[b], q_ref[...])
            dist = jnp.sum(xor, axis=1, keepdims=True)
            rows = lax.broadcasted_iota(jnp.int32, dist.shape, 0)
            combined = dist * _CAPACITY + (blk * _BLK + rows)
            best_ref[0] = jnp.minimum(best_ref[0], jnp.min(combined))

            @pl.when(k + _NBUF < _NBLK)
            def _start_next():
                _blk_start(keys_hbm, kbuf, sems, k + _NBUF, b)
        return 0

    lax.fori_loop(0, _NBLK // _NBUF, super_body, 0)

    idx = jnp.bitwise_and(best_ref[0], _CAPACITY - 1)
    copy = pltpu.make_async_copy(values_hbm.at[idx], out_ref, gsem)
    copy.start()
    copy.wait()


def kernel(query, keys, values):
    q2 = query.reshape(1, _KEY_BITS)
    return pl.pallas_call(
        _body,
        in_specs=[
            pl.BlockSpec(memory_space=pltpu.VMEM),
            pl.BlockSpec(memory_space=pltpu.MemorySpace.HBM),
            pl.BlockSpec(memory_space=pltpu.MemorySpace.HBM),
        ],
        out_specs=pl.BlockSpec(memory_space=pltpu.VMEM),
        out_shape=jax.ShapeDtypeStruct((_VALUE_BITS,), jnp.float32),
        scratch_shapes=[
            pltpu.VMEM((_NBUF, _BLK, _KEY_BITS), jnp.int32),
            pltpu.SMEM((1,), jnp.int32),
            pltpu.SemaphoreType.DMA,
        ]
        + [pltpu.SemaphoreType.DMA] * _NBUF,
    )(q2, keys, values)


# speedup vs baseline: 1.0100x; 1.0100x over previous
import jax
import jax.numpy as jnp
from jax import lax
from jax.experimental import pallas as pl
from jax.experimental.pallas import tpu as pltpu

_CAPACITY = 16384
_KEY_BITS = 2048
_VALUE_BITS = 2048
_BLK = 1024


def _body(q_ref, keys_ref, values_hbm, out_ref, best_ref, sem):
    i = pl.program_id(0)
    nblk = pl.num_programs(0)

    @pl.when(i == 0)
    def _init():
        best_ref[0] = jnp.int32(2**30)

    xor = jnp.bitwise_xor(keys_ref[...], q_ref[...])
    dist = jnp.sum(xor, axis=1, keepdims=True)
    rows = lax.broadcasted_iota(jnp.int32, dist.shape, 0)
    combined = dist * _CAPACITY + (i * _BLK + rows)
    best_ref[0] = jnp.minimum(best_ref[0], jnp.min(combined))

    @pl.when(i == nblk - 1)
    def _gather():
        idx = jnp.bitwise_and(best_ref[0], _CAPACITY - 1)
        copy = pltpu.make_async_copy(values_hbm.at[idx], out_ref, sem)
        copy.start()
        copy.wait()


def kernel(query, keys, values):
    q2 = query.reshape(1, _KEY_BITS)
    grid = _CAPACITY // _BLK
    return pl.pallas_call(
        _body,
        grid=(grid,),
        in_specs=[
            pl.BlockSpec((1, _KEY_BITS), lambda i: (0, 0)),
            pl.BlockSpec((_BLK, _KEY_BITS), lambda i: (i, 0)),
            pl.BlockSpec(memory_space=pltpu.MemorySpace.HBM),
        ],
        out_specs=pl.BlockSpec(memory_space=pltpu.VMEM),
        out_shape=jax.ShapeDtypeStruct((_VALUE_BITS,), jnp.float32),
        scratch_shapes=[
            pltpu.SMEM((1,), jnp.int32),
            pltpu.SemaphoreType.DMA,
        ],
    )(q2, keys, values)
